# baseline (speedup 1.0000x reference)
import functools

import jax
import jax.numpy as jnp
from jax import lax
from jax.experimental import pallas as pl
from jax.experimental.pallas import tpu as pltpu
from jax.experimental.pallas import tpu_sc as plsc

T = 4096
D = 1024
F = 512
E = 16
TOPK = 2
CAP = (T * TOPK // E) * 2
BLK = 256
NBLK = T * TOPK // BLK + E
ROWS = NBLK * BLK
TRASH = ROWS

NC = 2
NS = 16
NW = NC * NS


def _router_body(x_ref, rw_ref, dstw_ref, dstg_ref, w0x_ref, w1x_ref,
                 eblk_ref):
    x = x_ref[...]
    rw = rw_ref[...]
    logits = jnp.dot(x, rw, preferred_element_type=jnp.float32)

    eidx = lax.broadcasted_iota(jnp.int32, (T, E), 1)
    m0 = jnp.max(logits, axis=1, keepdims=True)
    e0 = jnp.min(jnp.where(logits == m0, eidx, E), axis=1)
    oh0 = eidx == e0[:, None]
    l1 = jnp.where(oh0, -jnp.inf, logits)
    m1 = jnp.max(l1, axis=1, keepdims=True)
    e1 = jnp.min(jnp.where(l1 == m1, eidx, E), axis=1)
    oh1 = eidx == e1[:, None]

    r = jnp.exp(m1[:, 0] - m0[:, 0])
    w0 = 1.0 / (1.0 + r)
    w1 = r / (1.0 + r)

    ohT = oh0.astype(jnp.float32) + oh1.astype(jnp.float32)
    BT = 512
    rows = lax.broadcasted_iota(jnp.int32, (BT, BT), 0)
    cols = lax.broadcasted_iota(jnp.int32, (BT, BT), 1)
    Lst = (rows > cols).astype(jnp.float32)
    base = jnp.zeros((1, E), jnp.float32)
    chunks = []
    for i in range(T // BT):
        blk = ohT[i * BT:(i + 1) * BT, :]
        chunks.append(jnp.dot(Lst, blk, preferred_element_type=jnp.float32)
                      + base)
        base = base + jnp.sum(blk, axis=0, keepdims=True)
    C = jnp.concatenate(chunks, axis=0)

    cnt_c = jnp.minimum(base, float(CAP))
    nb = jnp.floor((cnt_c + float(BLK - 1)) * (1.0 / BLK))
    r16 = lax.broadcasted_iota(jnp.int32, (E, E), 0)
    c16 = lax.broadcasted_iota(jnp.int32, (E, E), 1)
    U16 = (r16 < c16).astype(jnp.float32)
    base_blk = jnp.dot(nb, U16, preferred_element_type=jnp.float32)
    base_row = base_blk * float(BLK)
    base_row_b = jnp.broadcast_to(base_row, (T, E))

    biota = lax.broadcasted_iota(jnp.int32, (NBLK, E), 0)
    bcmp = biota >= jnp.broadcast_to(base_blk.astype(jnp.int32), (NBLK, E))
    eblk_ref[...] = jnp.sum(bcmp.astype(jnp.int32), axis=1) - 1

    pos0 = jnp.sum(jnp.where(oh0, C, 0.0), axis=1).astype(jnp.int32)
    pos1 = jnp.sum(jnp.where(oh1, C, 0.0), axis=1).astype(jnp.int32)
    br0 = jnp.sum(jnp.where(oh0, base_row_b, 0.0), axis=1).astype(jnp.int32)
    br1 = jnp.sum(jnp.where(oh1, base_row_b, 0.0), axis=1).astype(jnp.int32)

    v0 = pos0 < CAP
    v1 = pos1 < CAP
    dstw_ref[0, :] = jnp.where(v0, br0 + pos0, TRASH)
    dstw_ref[1, :] = jnp.where(v1, br1 + pos1, TRASH)
    dstg_ref[0, :] = br0 + jnp.where(v0, pos0, 0)
    dstg_ref[1, :] = br1 + jnp.where(v1, pos1, 0)
    w0m = jnp.where(v0, w0, 0.0)
    w1m = jnp.where(v1, w1, 0.0)
    w0x_ref[...] = jnp.broadcast_to(w0m[:, None], (T, E))
    w1x_ref[...] = jnp.broadcast_to(w1m[:, None], (T, E))


def _router(x, rw):
    return pl.pallas_call(
        _router_body,
        out_shape=(
            jax.ShapeDtypeStruct((2, T), jnp.int32),
            jax.ShapeDtypeStruct((2, T), jnp.int32),
            jax.ShapeDtypeStruct((T, E), jnp.float32),
            jax.ShapeDtypeStruct((T, E), jnp.float32),
            jax.ShapeDtypeStruct((NBLK,), jnp.int32),
        ),
    )(x, rw)


_DSUB = 32
_DCH = T // NW // _DSUB


def _dispatch(x, dstw8):
    mesh = plsc.VectorSubcoreMesh(core_axis_name="c", subcore_axis_name="s")

    @functools.partial(
        pl.kernel,
        out_type=jax.ShapeDtypeStruct((ROWS + 8, D // 2), jnp.int32),
        mesh=mesh,
        scratch_types=[
            pltpu.VMEM((_DSUB, D // 2), jnp.int32),
            pltpu.VMEM((_DSUB, D // 2), jnp.int32),
            pltpu.VMEM((_DCH, _DSUB), jnp.int32),
            pltpu.VMEM((_DCH, _DSUB), jnp.int32),
            pltpu.SemaphoreType.DMA,
            pltpu.SemaphoreType.DMA,
            pltpu.SemaphoreType.DMA,
            pltpu.SemaphoreType.DMA,
        ],
    )
    def k(x_hbm, dstw8_hbm, buf_hbm, rows0_v, rows1_v, idxa_v, idxb_v,
          sa0, sa1, sb0, sb1):
        rows = (rows0_v, rows1_v)
        sa = (sa0, sa1)
        sb = (sb0, sb1)
        cid = lax.axis_index("c")
        sid = lax.axis_index("s")
        wid = sid * NC + cid
        tb = wid * (T // NW)
        pltpu.sync_copy(dstw8_hbm.at[0, wid], idxa_v)
        pltpu.sync_copy(dstw8_hbm.at[1, wid], idxb_v)
        pltpu.sync_copy(x_hbm.at[pl.ds(tb, _DSUB)], rows[0])
        for c in range(_DCH):
            slot = c % 2
            cpa = pltpu.async_copy(rows[slot], buf_hbm.at[idxa_v.at[c]],
                                   sa[slot])
            cpb = pltpu.async_copy(rows[slot], buf_hbm.at[idxb_v.at[c]],
                                   sb[slot])
            if c + 1 < _DCH:
                pltpu.sync_copy(
                    x_hbm.at[pl.ds(tb + (c + 1) * _DSUB, _DSUB)],
                    rows[(c + 1) % 2])
            cpa.wait()
            cpb.wait()

    return k(x, dstw8)


_M16 = -65536


def _mlp_body(eblk_ref, buf_ref, wg_ref, wu_ref, wd_ref, out_ref):
    v = buf_ref[...]
    lo = lax.bitcast_convert_type(v << 16, jnp.float32)
    hi = lax.bitcast_convert_type(v & _M16, jnp.float32)
    xb = jnp.concatenate([lo, hi], axis=1).astype(jnp.bfloat16)
    wg = wg_ref[0].astype(jnp.bfloat16)
    wu = wu_ref[0].astype(jnp.bfloat16)
    wd = wd_ref[0].astype(jnp.bfloat16)
    g = jnp.dot(xb, wg, preferred_element_type=jnp.float32)
    u = jnp.dot(xb, wu, preferred_element_type=jnp.float32)
    h = ((g * jax.nn.sigmoid(g)) * u).astype(jnp.bfloat16)
    y = jnp.dot(h, wd, preferred_element_type=jnp.float32)
    ya = lax.bitcast_convert_type(
        y[:, :D // 2].astype(jnp.bfloat16).astype(jnp.float32), jnp.int32)
    yb = lax.bitcast_convert_type(
        y[:, D // 2:].astype(jnp.bfloat16).astype(jnp.float32), jnp.int32)
    out_ref[...] = yb | lax.shift_right_logical(ya, 16)


def _mlp(eblk, buf, w_gate, w_up, w_down):
    grid_spec = pltpu.PrefetchScalarGridSpec(
        num_scalar_prefetch=1,
        grid=(NBLK,),
        in_specs=[
            pl.BlockSpec((BLK, D // 2), lambda b, eb: (b, 0)),
            pl.BlockSpec((1, D, F), lambda b, eb: (eb[b], 0, 0)),
            pl.BlockSpec((1, D, F), lambda b, eb: (eb[b], 0, 0)),
            pl.BlockSpec((1, F, D), lambda b, eb: (eb[b], 0, 0)),
        ],
        out_specs=pl.BlockSpec((BLK, D // 2), lambda b, eb: (b, 0)),
    )
    return pl.pallas_call(
        _mlp_body,
        grid_spec=grid_spec,
        out_shape=jax.ShapeDtypeStruct((ROWS, D // 2), jnp.int32),
    )(eblk, buf, w_gate, w_up, w_down)


_CSUB = 16
_CSTEPS = T // NW // _CSUB
_TPW = T // NW


def _combine(out_buf, dstg, w0x, w1x):
    mesh = plsc.VectorSubcoreMesh(core_axis_name="c", subcore_axis_name="s")

    @functools.partial(
        pl.kernel,
        out_type=jax.ShapeDtypeStruct((T, D), jnp.float32),
        mesh=mesh,
        scratch_types=[
            pltpu.VMEM((_CSUB, D // 2), jnp.int32),
            pltpu.VMEM((_CSUB, D // 2), jnp.int32),
            pltpu.VMEM((_CSUB, D // 2), jnp.int32),
            pltpu.VMEM((_CSUB, D // 2), jnp.int32),
            pltpu.VMEM((_CSUB, D), jnp.float32),
            pltpu.VMEM((_TPW,), jnp.int32),
            pltpu.VMEM((_TPW,), jnp.int32),
            pltpu.VMEM((_TPW, E), jnp.float32),
            pltpu.VMEM((_TPW, E), jnp.float32),
            pltpu.SemaphoreType.DMA,
            pltpu.SemaphoreType.DMA,
            pltpu.SemaphoreType.DMA,
            pltpu.SemaphoreType.DMA,
            pltpu.SemaphoreType.DMA,
        ],
    )
    def k(out_hbm, dstg_hbm, w0x_hbm, w1x_hbm, y_hbm,
          a0_v, a1_v, b0_v, b1_v, y_v, i0_v, i1_v, w0_v, w1_v,
          sa0, sa1, sb0, sb1, sy):
        a_v = (a0_v, a1_v)
        b_v = (b0_v, b1_v)
        sa = (sa0, sa1)
        sb = (sb0, sb1)
        cid = lax.axis_index("c")
        sid = lax.axis_index("s")
        wid = sid * NC + cid
        tb = wid * _TPW
        pltpu.sync_copy(dstg_hbm.at[0, pl.ds(tb, _TPW)], i0_v)
        pltpu.sync_copy(dstg_hbm.at[1, pl.ds(tb, _TPW)], i1_v)
        pltpu.sync_copy(w0x_hbm.at[pl.ds(tb, _TPW)], w0_v)
        pltpu.sync_copy(w1x_hbm.at[pl.ds(tb, _TPW)], w1_v)

        def gather(c):
            slot = c % 2
            sl = pl.ds(c * _CSUB, _CSUB)
            ca = pltpu.async_copy(out_hbm.at[i0_v.at[sl]], a_v[slot], sa[slot])
            cb = pltpu.async_copy(out_hbm.at[i1_v.at[sl]], b_v[slot], sb[slot])
            return ca, cb

        pend = gather(0)
        ystore = None
        for c in range(_CSTEPS):
            slot = c % 2
            nxt = gather(c + 1) if c + 1 < _CSTEPS else None
            pend[0].wait()
            pend[1].wait()
            if ystore is not None:
                ystore.wait()

            def tok(j, _):
                wa = w0_v[c * _CSUB + j, :]
                wb = w1_v[c * _CSUB + j, :]
                for v in range(D // 2 // 16):
                    sl = pl.ds(v * 16, 16)
                    sh = pl.ds(D // 2 + v * 16, 16)
                    a = a_v[slot][j, sl]
                    b = b_v[slot][j, sl]
                    alo = lax.bitcast_convert_type(a << 16, jnp.float32)
                    blo = lax.bitcast_convert_type(b << 16, jnp.float32)
                    ahi = lax.bitcast_convert_type(a & _M16, jnp.float32)
                    bhi = lax.bitcast_convert_type(b & _M16, jnp.float32)
                    y_v[j, sl] = alo * wa + blo * wb
                    y_v[j, sh] = ahi * wa + bhi * wb
                return 0

            lax.fori_loop(0, _CSUB, tok, 0)
            ystore = pltpu.async_copy(
                y_v, y_hbm.at[pl.ds(tb + c * _CSUB, _CSUB)], sy)
            pend = nxt
        ystore.wait()

    return k(out_buf, dstg, w0x, w1x)


def kernel(hidden_states, router_w, w_gate, w_up, w_down):
    dstw, dstg, w0x, w1x, eblk = _router(hidden_states, router_w)
    dstw8 = dstw.reshape(2, NW, _DCH, _DSUB)
    x16 = hidden_states.astype(jnp.bfloat16)
    lo = lax.bitcast_convert_type(x16[:, :D // 2], jnp.uint16)
    hi = lax.bitcast_convert_type(x16[:, D // 2:], jnp.uint16)
    xp = lax.bitcast_convert_type(
        (hi.astype(jnp.uint32) << 16) | lo.astype(jnp.uint32), jnp.int32)
    buf = _dispatch(xp, dstw8)
    out_buf = _mlp(eblk, buf, w_gate, w_up, w_down)
    return _combine(out_buf, dstg, w0x, w1x)


# speedup vs baseline: 1.0448x; 1.0448x over previous
import functools

import jax
import jax.numpy as jnp
from jax import lax
from jax.experimental import pallas as pl
from jax.experimental.pallas import tpu as pltpu
from jax.experimental.pallas import tpu_sc as plsc

T = 4096
D = 1024
F = 512
E = 16
TOPK = 2
CAP = (T * TOPK // E) * 2
BLK = 256
NBLK = T * TOPK // BLK + E
ROWS = NBLK * BLK
TRASH = ROWS

NC = 2
NS = 16
NW = NC * NS


def _router_body(x_ref, rw_ref, dstw_ref, dstg_ref, w0x_ref, w1x_ref,
                 eblk_ref):
    x = x_ref[...]
    rw = rw_ref[...]
    logits = jnp.dot(x, rw, preferred_element_type=jnp.float32)

    eidx = lax.broadcasted_iota(jnp.int32, (T, E), 1)
    m0 = jnp.max(logits, axis=1, keepdims=True)
    e0 = jnp.min(jnp.where(logits == m0, eidx, E), axis=1)
    oh0 = eidx == e0[:, None]
    l1 = jnp.where(oh0, -jnp.inf, logits)
    m1 = jnp.max(l1, axis=1, keepdims=True)
    e1 = jnp.min(jnp.where(l1 == m1, eidx, E), axis=1)
    oh1 = eidx == e1[:, None]

    r = jnp.exp(m1[:, 0] - m0[:, 0])
    w0 = 1.0 / (1.0 + r)
    w1 = r / (1.0 + r)

    ohT = oh0.astype(jnp.float32) + oh1.astype(jnp.float32)
    BT = 512
    rows = lax.broadcasted_iota(jnp.int32, (BT, BT), 0)
    cols = lax.broadcasted_iota(jnp.int32, (BT, BT), 1)
    Lst = (rows > cols).astype(jnp.float32)
    base = jnp.zeros((1, E), jnp.float32)
    chunks = []
    for i in range(T // BT):
        blk = ohT[i * BT:(i + 1) * BT, :]
        chunks.append(jnp.dot(Lst, blk, preferred_element_type=jnp.float32)
                      + base)
        base = base + jnp.sum(blk, axis=0, keepdims=True)
    C = jnp.concatenate(chunks, axis=0)

    cnt_c = jnp.minimum(base, float(CAP))
    nb = jnp.floor((cnt_c + float(BLK - 1)) * (1.0 / BLK))
    r16 = lax.broadcasted_iota(jnp.int32, (E, E), 0)
    c16 = lax.broadcasted_iota(jnp.int32, (E, E), 1)
    U16 = (r16 < c16).astype(jnp.float32)
    base_blk = jnp.dot(nb, U16, preferred_element_type=jnp.float32)
    base_row = base_blk * float(BLK)
    base_row_b = jnp.broadcast_to(base_row, (T, E))

    biota = lax.broadcasted_iota(jnp.int32, (NBLK, E), 0)
    bcmp = biota >= jnp.broadcast_to(base_blk.astype(jnp.int32), (NBLK, E))
    eblk_ref[...] = jnp.sum(bcmp.astype(jnp.int32), axis=1) - 1

    pos0 = jnp.sum(jnp.where(oh0, C, 0.0), axis=1).astype(jnp.int32)
    pos1 = jnp.sum(jnp.where(oh1, C, 0.0), axis=1).astype(jnp.int32)
    br0 = jnp.sum(jnp.where(oh0, base_row_b, 0.0), axis=1).astype(jnp.int32)
    br1 = jnp.sum(jnp.where(oh1, base_row_b, 0.0), axis=1).astype(jnp.int32)

    v0 = pos0 < CAP
    v1 = pos1 < CAP
    dstw_ref[0, :] = jnp.where(v0, br0 + pos0, TRASH)
    dstw_ref[1, :] = jnp.where(v1, br1 + pos1, TRASH)
    dstg_ref[0, :] = br0 + jnp.where(v0, pos0, 0)
    dstg_ref[1, :] = br1 + jnp.where(v1, pos1, 0)
    w0m = jnp.where(v0, w0, 0.0)
    w1m = jnp.where(v1, w1, 0.0)
    w0x_ref[...] = jnp.broadcast_to(w0m[:, None], (T, E))
    w1x_ref[...] = jnp.broadcast_to(w1m[:, None], (T, E))


def _router(x, rw):
    return pl.pallas_call(
        _router_body,
        out_shape=(
            jax.ShapeDtypeStruct((2, T), jnp.int32),
            jax.ShapeDtypeStruct((2, T), jnp.int32),
            jax.ShapeDtypeStruct((T, E), jnp.float32),
            jax.ShapeDtypeStruct((T, E), jnp.float32),
            jax.ShapeDtypeStruct((NBLK,), jnp.int32),
        ),
    )(x, rw)


_DSUB = 32
_DCH = T // NW // _DSUB


def _dispatch(x, dstw8):
    mesh = plsc.VectorSubcoreMesh(core_axis_name="c", subcore_axis_name="s")

    @functools.partial(
        pl.kernel,
        out_type=jax.ShapeDtypeStruct((ROWS + 8, D // 2), jnp.int32),
        mesh=mesh,
        scratch_types=[
            pltpu.VMEM((_DSUB, D // 2), jnp.int32),
            pltpu.VMEM((_DSUB, D // 2), jnp.int32),
            pltpu.VMEM((_DCH, _DSUB), jnp.int32),
            pltpu.VMEM((_DCH, _DSUB), jnp.int32),
            pltpu.SemaphoreType.DMA,
            pltpu.SemaphoreType.DMA,
            pltpu.SemaphoreType.DMA,
            pltpu.SemaphoreType.DMA,
        ],
    )
    def k(x_hbm, dstw8_hbm, buf_hbm, rows0_v, rows1_v, idxa_v, idxb_v,
          sa0, sa1, sb0, sb1):
        rows = (rows0_v, rows1_v)
        sa = (sa0, sa1)
        sb = (sb0, sb1)
        cid = lax.axis_index("c")
        sid = lax.axis_index("s")
        wid = sid * NC + cid
        tb = wid * (T // NW)
        pltpu.sync_copy(dstw8_hbm.at[0, wid], idxa_v)
        pltpu.sync_copy(dstw8_hbm.at[1, wid], idxb_v)
        pltpu.sync_copy(x_hbm.at[pl.ds(tb, _DSUB)], rows[0])
        for c in range(_DCH):
            slot = c % 2
            cpa = pltpu.async_copy(rows[slot], buf_hbm.at[idxa_v.at[c]],
                                   sa[slot])
            cpb = pltpu.async_copy(rows[slot], buf_hbm.at[idxb_v.at[c]],
                                   sb[slot])
            if c + 1 < _DCH:
                pltpu.sync_copy(
                    x_hbm.at[pl.ds(tb + (c + 1) * _DSUB, _DSUB)],
                    rows[(c + 1) % 2])
            cpa.wait()
            cpb.wait()

    return k(x, dstw8)


_M16 = -65536


def _mlp_body(eblk_ref, buf_ref, wg_ref, wu_ref, wd_ref, out_ref):
    v = buf_ref[...]
    lo = lax.bitcast_convert_type(v << 16, jnp.float32)
    hi = lax.bitcast_convert_type(v & _M16, jnp.float32)
    xb = jnp.concatenate([lo, hi], axis=1).astype(jnp.bfloat16)
    wg = wg_ref[0].astype(jnp.bfloat16)
    wu = wu_ref[0].astype(jnp.bfloat16)
    wd = wd_ref[0].astype(jnp.bfloat16)
    g = jnp.dot(xb, wg, preferred_element_type=jnp.float32)
    u = jnp.dot(xb, wu, preferred_element_type=jnp.float32)
    h = ((g * jax.nn.sigmoid(g)) * u).astype(jnp.bfloat16)
    out_ref[...] = jnp.dot(h, wd, preferred_element_type=jnp.float32)


def _mlp(eblk, buf, w_gate, w_up, w_down):
    grid_spec = pltpu.PrefetchScalarGridSpec(
        num_scalar_prefetch=1,
        grid=(NBLK,),
        in_specs=[
            pl.BlockSpec((BLK, D // 2), lambda b, eb: (b, 0)),
            pl.BlockSpec((1, D, F), lambda b, eb: (eb[b], 0, 0)),
            pl.BlockSpec((1, D, F), lambda b, eb: (eb[b], 0, 0)),
            pl.BlockSpec((1, F, D), lambda b, eb: (eb[b], 0, 0)),
        ],
        out_specs=pl.BlockSpec((BLK, D), lambda b, eb: (b, 0)),
    )
    return pl.pallas_call(
        _mlp_body,
        grid_spec=grid_spec,
        out_shape=jax.ShapeDtypeStruct((ROWS, D), jnp.float32),
    )(eblk, buf, w_gate, w_up, w_down)


_CSUB = 16
_CSTEPS = T // NW // _CSUB
_TPW = T // NW


def _combine(out_buf, dstg, w0x, w1x):
    mesh = plsc.VectorSubcoreMesh(core_axis_name="c", subcore_axis_name="s")

    @functools.partial(
        pl.kernel,
        out_type=jax.ShapeDtypeStruct((T, D), jnp.float32),
        mesh=mesh,
        scratch_types=[
            pltpu.VMEM((_CSUB, D), jnp.float32),
            pltpu.VMEM((_CSUB, D), jnp.float32),
            pltpu.VMEM((_CSUB, D), jnp.float32),
            pltpu.VMEM((_CSUB, D), jnp.float32),
            pltpu.VMEM((_CSUB, D), jnp.float32),
            pltpu.VMEM((_TPW,), jnp.int32),
            pltpu.VMEM((_TPW,), jnp.int32),
            pltpu.VMEM((_TPW, E), jnp.float32),
            pltpu.VMEM((_TPW, E), jnp.float32),
            pltpu.SemaphoreType.DMA,
            pltpu.SemaphoreType.DMA,
            pltpu.SemaphoreType.DMA,
            pltpu.SemaphoreType.DMA,
            pltpu.SemaphoreType.DMA,
        ],
    )
    def k(out_hbm, dstg_hbm, w0x_hbm, w1x_hbm, y_hbm,
          a0_v, a1_v, b0_v, b1_v, y_v, i0_v, i1_v, w0_v, w1_v,
          sa0, sa1, sb0, sb1, sy):
        a_v = (a0_v, a1_v)
        b_v = (b0_v, b1_v)
        sa = (sa0, sa1)
        sb = (sb0, sb1)
        cid = lax.axis_index("c")
        sid = lax.axis_index("s")
        wid = sid * NC + cid
        tb = wid * _TPW
        pltpu.sync_copy(dstg_hbm.at[0, pl.ds(tb, _TPW)], i0_v)
        pltpu.sync_copy(dstg_hbm.at[1, pl.ds(tb, _TPW)], i1_v)
        pltpu.sync_copy(w0x_hbm.at[pl.ds(tb, _TPW)], w0_v)
        pltpu.sync_copy(w1x_hbm.at[pl.ds(tb, _TPW)], w1_v)

        def gather(c):
            slot = c % 2
            sl = pl.ds(c * _CSUB, _CSUB)
            ca = pltpu.async_copy(out_hbm.at[i0_v.at[sl]], a_v[slot], sa[slot])
            cb = pltpu.async_copy(out_hbm.at[i1_v.at[sl]], b_v[slot], sb[slot])
            return ca, cb

        pend = gather(0)
        ystore = None
        for c in range(_CSTEPS):
            slot = c % 2
            nxt = gather(c + 1) if c + 1 < _CSTEPS else None
            pend[0].wait()
            pend[1].wait()
            if ystore is not None:
                ystore.wait()

            def tok(j, _):
                wa = w0_v[c * _CSUB + j, :]
                wb = w1_v[c * _CSUB + j, :]
                for v in range(D // 16):
                    sl = pl.ds(v * 16, 16)
                    y_v[j, sl] = (a_v[slot][j, sl] * wa
                                  + b_v[slot][j, sl] * wb)
                return 0

            lax.fori_loop(0, _CSUB, tok, 0)
            ystore = pltpu.async_copy(
                y_v, y_hbm.at[pl.ds(tb + c * _CSUB, _CSUB)], sy)
            pend = nxt
        ystore.wait()

    return k(out_buf, dstg, w0x, w1x)


def kernel(hidden_states, router_w, w_gate, w_up, w_down):
    dstw, dstg, w0x, w1x, eblk = _router(hidden_states, router_w)
    dstw8 = dstw.reshape(2, NW, _DCH, _DSUB)
    x16 = hidden_states.astype(jnp.bfloat16)
    lo = lax.bitcast_convert_type(x16[:, :D // 2], jnp.uint16)
    hi = lax.bitcast_convert_type(x16[:, D // 2:], jnp.uint16)
    xp = lax.bitcast_convert_type(
        (hi.astype(jnp.uint32) << 16) | lo.astype(jnp.uint32), jnp.int32)
    buf = _dispatch(xp, dstw8)
    out_buf = _mlp(eblk, buf, w_gate, w_up, w_down)
    return _combine(out_buf, dstg, w0x, w1x)


# speedup vs baseline: 1.0731x; 1.0271x over previous
import functools

import jax
import jax.numpy as jnp
from jax import lax
from jax.experimental import pallas as pl
from jax.experimental.pallas import tpu as pltpu
from jax.experimental.pallas import tpu_sc as plsc

T = 4096
D = 1024
F = 512
E = 16
TOPK = 2
CAP = (T * TOPK // E) * 2
BLK = 256
NBLK = T * TOPK // BLK + E
ROWS = NBLK * BLK
TRASH = ROWS

NC = 2
NS = 16
NW = NC * NS


def _router_body(x_ref, rw_ref, dstw_ref, dstg_ref, w0x_ref, w1x_ref,
                 eblk_ref):
    x = x_ref[...]
    rw = rw_ref[...]
    logits = jnp.dot(x, rw, preferred_element_type=jnp.float32)

    eidx = lax.broadcasted_iota(jnp.int32, (T, E), 1)
    m0 = jnp.max(logits, axis=1, keepdims=True)
    e0 = jnp.min(jnp.where(logits == m0, eidx, E), axis=1)
    oh0 = eidx == e0[:, None]
    l1 = jnp.where(oh0, -jnp.inf, logits)
    m1 = jnp.max(l1, axis=1, keepdims=True)
    e1 = jnp.min(jnp.where(l1 == m1, eidx, E), axis=1)
    oh1 = eidx == e1[:, None]

    r = jnp.exp(m1[:, 0] - m0[:, 0])
    w0 = 1.0 / (1.0 + r)
    w1 = r / (1.0 + r)

    ohT = oh0.astype(jnp.float32) + oh1.astype(jnp.float32)
    BT = 512
    rows = lax.broadcasted_iota(jnp.int32, (BT, BT), 0)
    cols = lax.broadcasted_iota(jnp.int32, (BT, BT), 1)
    Lst = (rows > cols).astype(jnp.float32)
    base = jnp.zeros((1, E), jnp.float32)
    chunks = []
    for i in range(T // BT):
        blk = ohT[i * BT:(i + 1) * BT, :]
        chunks.append(jnp.dot(Lst, blk, preferred_element_type=jnp.float32)
                      + base)
        base = base + jnp.sum(blk, axis=0, keepdims=True)
    C = jnp.concatenate(chunks, axis=0)

    cnt_c = jnp.minimum(base, float(CAP))
    nb = jnp.floor((cnt_c + float(BLK - 1)) * (1.0 / BLK))
    r16 = lax.broadcasted_iota(jnp.int32, (E, E), 0)
    c16 = lax.broadcasted_iota(jnp.int32, (E, E), 1)
    U16 = (r16 < c16).astype(jnp.float32)
    base_blk = jnp.dot(nb, U16, preferred_element_type=jnp.float32)
    base_row = base_blk * float(BLK)
    base_row_b = jnp.broadcast_to(base_row, (T, E))

    biota = lax.broadcasted_iota(jnp.int32, (NBLK, E), 0)
    bcmp = biota >= jnp.broadcast_to(base_blk.astype(jnp.int32), (NBLK, E))
    eblk_ref[...] = jnp.sum(bcmp.astype(jnp.int32), axis=1) - 1

    pos0 = jnp.sum(jnp.where(oh0, C, 0.0), axis=1).astype(jnp.int32)
    pos1 = jnp.sum(jnp.where(oh1, C, 0.0), axis=1).astype(jnp.int32)
    br0 = jnp.sum(jnp.where(oh0, base_row_b, 0.0), axis=1).astype(jnp.int32)
    br1 = jnp.sum(jnp.where(oh1, base_row_b, 0.0), axis=1).astype(jnp.int32)

    v0 = pos0 < CAP
    v1 = pos1 < CAP
    dstw_ref[0, :] = jnp.where(v0, br0 + pos0, TRASH)
    dstw_ref[1, :] = jnp.where(v1, br1 + pos1, TRASH)
    dstg_ref[0, :] = br0 + jnp.where(v0, pos0, 0)
    dstg_ref[1, :] = br1 + jnp.where(v1, pos1, 0)
    w0m = jnp.where(v0, w0, 0.0)
    w1m = jnp.where(v1, w1, 0.0)
    w0x_ref[...] = jnp.broadcast_to(w0m[:, None], (T, E))
    w1x_ref[...] = jnp.broadcast_to(w1m[:, None], (T, E))


def _router(x, rw):
    return pl.pallas_call(
        _router_body,
        out_shape=(
            jax.ShapeDtypeStruct((2, T), jnp.int32),
            jax.ShapeDtypeStruct((2, T), jnp.int32),
            jax.ShapeDtypeStruct((T, E), jnp.float32),
            jax.ShapeDtypeStruct((T, E), jnp.float32),
            jax.ShapeDtypeStruct((NBLK,), jnp.int32),
        ),
    )(x, rw)


_DSUB = 32
_DCH = T // NW // _DSUB


def _dispatch(x, dstw8):
    mesh = plsc.VectorSubcoreMesh(core_axis_name="c", subcore_axis_name="s")

    @functools.partial(
        pl.kernel,
        out_type=jax.ShapeDtypeStruct((ROWS + 8, D), jnp.float32),
        mesh=mesh,
        scratch_types=[
            pltpu.VMEM((_DSUB, D), jnp.float32),
            pltpu.VMEM((_DSUB, D), jnp.float32),
            pltpu.VMEM((_DCH, _DSUB), jnp.int32),
            pltpu.VMEM((_DCH, _DSUB), jnp.int32),
            pltpu.SemaphoreType.DMA,
            pltpu.SemaphoreType.DMA,
            pltpu.SemaphoreType.DMA,
            pltpu.SemaphoreType.DMA,
        ],
    )
    def k(x_hbm, dstw8_hbm, buf_hbm, rows0_v, rows1_v, idxa_v, idxb_v,
          sa0, sa1, sb0, sb1):
        rows = (rows0_v, rows1_v)
        sa = (sa0, sa1)
        sb = (sb0, sb1)
        cid = lax.axis_index("c")
        sid = lax.axis_index("s")
        wid = sid * NC + cid
        tb = wid * (T // NW)
        pltpu.sync_copy(dstw8_hbm.at[0, wid], idxa_v)
        pltpu.sync_copy(dstw8_hbm.at[1, wid], idxb_v)
        pltpu.sync_copy(x_hbm.at[pl.ds(tb, _DSUB)], rows[0])
        for c in range(_DCH):
            slot = c % 2
            cpa = pltpu.async_copy(rows[slot], buf_hbm.at[idxa_v.at[c]],
                                   sa[slot])
            cpb = pltpu.async_copy(rows[slot], buf_hbm.at[idxb_v.at[c]],
                                   sb[slot])
            if c + 1 < _DCH:
                pltpu.sync_copy(
                    x_hbm.at[pl.ds(tb + (c + 1) * _DSUB, _DSUB)],
                    rows[(c + 1) % 2])
            cpa.wait()
            cpb.wait()

    return k(x, dstw8)


def _mlp_body(eblk_ref, buf_ref, wg_ref, wu_ref, wd_ref, out_ref):
    xb = buf_ref[...].astype(jnp.bfloat16)
    wg = wg_ref[0].astype(jnp.bfloat16)
    wu = wu_ref[0].astype(jnp.bfloat16)
    wd = wd_ref[0].astype(jnp.bfloat16)
    g = jnp.dot(xb, wg, preferred_element_type=jnp.float32)
    u = jnp.dot(xb, wu, preferred_element_type=jnp.float32)
    h = ((g * jax.nn.sigmoid(g)) * u).astype(jnp.bfloat16)
    out_ref[...] = jnp.dot(h, wd, preferred_element_type=jnp.float32)


def _mlp(eblk, buf, w_gate, w_up, w_down):
    grid_spec = pltpu.PrefetchScalarGridSpec(
        num_scalar_prefetch=1,
        grid=(NBLK,),
        in_specs=[
            pl.BlockSpec((BLK, D), lambda b, eb: (b, 0)),
            pl.BlockSpec((1, D, F), lambda b, eb: (eb[b], 0, 0)),
            pl.BlockSpec((1, D, F), lambda b, eb: (eb[b], 0, 0)),
            pl.BlockSpec((1, F, D), lambda b, eb: (eb[b], 0, 0)),
        ],
        out_specs=pl.BlockSpec((BLK, D), lambda b, eb: (b, 0)),
    )
    return pl.pallas_call(
        _mlp_body,
        grid_spec=grid_spec,
        out_shape=jax.ShapeDtypeStruct((ROWS, D), jnp.float32),
    )(eblk, buf, w_gate, w_up, w_down)


_CSUB = 16
_CSTEPS = T // NW // _CSUB
_TPW = T // NW


def _combine(out_buf, dstg, w0x, w1x):
    mesh = plsc.VectorSubcoreMesh(core_axis_name="c", subcore_axis_name="s")

    @functools.partial(
        pl.kernel,
        out_type=jax.ShapeDtypeStruct((T, D), jnp.float32),
        mesh=mesh,
        scratch_types=[
            pltpu.VMEM((_CSUB, D), jnp.float32),
            pltpu.VMEM((_CSUB, D), jnp.float32),
            pltpu.VMEM((_CSUB, D), jnp.float32),
            pltpu.VMEM((_CSUB, D), jnp.float32),
            pltpu.VMEM((_CSUB, D), jnp.float32),
            pltpu.VMEM((_TPW,), jnp.int32),
            pltpu.VMEM((_TPW,), jnp.int32),
            pltpu.VMEM((_TPW, E), jnp.float32),
            pltpu.VMEM((_TPW, E), jnp.float32),
            pltpu.SemaphoreType.DMA,
            pltpu.SemaphoreType.DMA,
            pltpu.SemaphoreType.DMA,
            pltpu.SemaphoreType.DMA,
            pltpu.SemaphoreType.DMA,
        ],
    )
    def k(out_hbm, dstg_hbm, w0x_hbm, w1x_hbm, y_hbm,
          a0_v, a1_v, b0_v, b1_v, y_v, i0_v, i1_v, w0_v, w1_v,
          sa0, sa1, sb0, sb1, sy):
        a_v = (a0_v, a1_v)
        b_v = (b0_v, b1_v)
        sa = (sa0, sa1)
        sb = (sb0, sb1)
        cid = lax.axis_index("c")
        sid = lax.axis_index("s")
        wid = sid * NC + cid
        tb = wid * _TPW
        pltpu.sync_copy(dstg_hbm.at[0, pl.ds(tb, _TPW)], i0_v)
        pltpu.sync_copy(dstg_hbm.at[1, pl.ds(tb, _TPW)], i1_v)
        pltpu.sync_copy(w0x_hbm.at[pl.ds(tb, _TPW)], w0_v)
        pltpu.sync_copy(w1x_hbm.at[pl.ds(tb, _TPW)], w1_v)

        def gather(c):
            slot = c % 2
            sl = pl.ds(c * _CSUB, _CSUB)
            ca = pltpu.async_copy(out_hbm.at[i0_v.at[sl]], a_v[slot], sa[slot])
            cb = pltpu.async_copy(out_hbm.at[i1_v.at[sl]], b_v[slot], sb[slot])
            return ca, cb

        pend = gather(0)
        ystore = None
        for c in range(_CSTEPS):
            slot = c % 2
            nxt = gather(c + 1) if c + 1 < _CSTEPS else None
            pend[0].wait()
            pend[1].wait()
            if ystore is not None:
                ystore.wait()

            def tok(j, _):
                wa = w0_v[c * _CSUB + j, :]
                wb = w1_v[c * _CSUB + j, :]
                for v in range(D // 16):
                    sl = pl.ds(v * 16, 16)
                    y_v[j, sl] = (a_v[slot][j, sl] * wa
                                  + b_v[slot][j, sl] * wb)
                return 0

            lax.fori_loop(0, _CSUB, tok, 0)
            ystore = pltpu.async_copy(
                y_v, y_hbm.at[pl.ds(tb + c * _CSUB, _CSUB)], sy)
            pend = nxt
        ystore.wait()

    return k(out_buf, dstg, w0x, w1x)


def kernel(hidden_states, router_w, w_gate, w_up, w_down):
    dstw, dstg, w0x, w1x, eblk = _router(hidden_states, router_w)
    dstw8 = dstw.reshape(2, NW, _DCH, _DSUB)
    buf = _dispatch(hidden_states, dstw8)
    out_buf = _mlp(eblk, buf, w_gate, w_up, w_down)
    return _combine(out_buf, dstg, w0x, w1x)


# speedup vs baseline: 1.0777x; 1.0042x over previous
import functools

import jax
import jax.numpy as jnp
from jax import lax
from jax.experimental import pallas as pl
from jax.experimental.pallas import tpu as pltpu
from jax.experimental.pallas import tpu_sc as plsc

T = 4096
D = 1024
F = 512
E = 16
TOPK = 2
CAP = (T * TOPK // E) * 2
BLK = 256
NBLK = T * TOPK // BLK + E
ROWS = NBLK * BLK
TRASH = ROWS

NC = 2
NS = 16
NW = NC * NS


def _router_body(x_ref, rw_ref, dstw_ref, dstg_ref, w0x_ref, w1x_ref,
                 eblk_ref):
    x = x_ref[...]
    rw = rw_ref[...]
    logits = jnp.dot(x, rw, preferred_element_type=jnp.float32)

    eidx = lax.broadcasted_iota(jnp.int32, (T, E), 1)
    m0 = jnp.max(logits, axis=1, keepdims=True)
    e0 = jnp.min(jnp.where(logits == m0, eidx, E), axis=1)
    oh0 = eidx == e0[:, None]
    l1 = jnp.where(oh0, -jnp.inf, logits)
    m1 = jnp.max(l1, axis=1, keepdims=True)
    e1 = jnp.min(jnp.where(l1 == m1, eidx, E), axis=1)
    oh1 = eidx == e1[:, None]

    r = jnp.exp(m1[:, 0] - m0[:, 0])
    w0 = 1.0 / (1.0 + r)
    w1 = r / (1.0 + r)

    ohT = oh0.astype(jnp.float32) + oh1.astype(jnp.float32)
    BT = 512
    rows = lax.broadcasted_iota(jnp.int32, (BT, BT), 0)
    cols = lax.broadcasted_iota(jnp.int32, (BT, BT), 1)
    Lst = (rows > cols).astype(jnp.float32)
    base = jnp.zeros((1, E), jnp.float32)
    chunks = []
    for i in range(T // BT):
        blk = ohT[i * BT:(i + 1) * BT, :]
        chunks.append(jnp.dot(Lst, blk, preferred_element_type=jnp.float32)
                      + base)
        base = base + jnp.sum(blk, axis=0, keepdims=True)
    C = jnp.concatenate(chunks, axis=0)

    cnt_c = jnp.minimum(base, float(CAP))
    nb = jnp.floor((cnt_c + float(BLK - 1)) * (1.0 / BLK))
    r16 = lax.broadcasted_iota(jnp.int32, (E, E), 0)
    c16 = lax.broadcasted_iota(jnp.int32, (E, E), 1)
    U16 = (r16 < c16).astype(jnp.float32)
    base_blk = jnp.dot(nb, U16, preferred_element_type=jnp.float32)
    base_row = base_blk * float(BLK)
    base_row_b = jnp.broadcast_to(base_row, (T, E))

    biota = lax.broadcasted_iota(jnp.int32, (NBLK, E), 0)
    bcmp = biota >= jnp.broadcast_to(base_blk.astype(jnp.int32), (NBLK, E))
    eblk_ref[...] = jnp.sum(bcmp.astype(jnp.int32), axis=1) - 1

    pos0 = jnp.sum(jnp.where(oh0, C, 0.0), axis=1).astype(jnp.int32)
    pos1 = jnp.sum(jnp.where(oh1, C, 0.0), axis=1).astype(jnp.int32)
    br0 = jnp.sum(jnp.where(oh0, base_row_b, 0.0), axis=1).astype(jnp.int32)
    br1 = jnp.sum(jnp.where(oh1, base_row_b, 0.0), axis=1).astype(jnp.int32)

    v0 = pos0 < CAP
    v1 = pos1 < CAP
    dstw_ref[0, :] = jnp.where(v0, br0 + pos0, TRASH)
    dstw_ref[1, :] = jnp.where(v1, br1 + pos1, TRASH)
    dstg_ref[0, :] = br0 + jnp.where(v0, pos0, 0)
    dstg_ref[1, :] = br1 + jnp.where(v1, pos1, 0)
    w0m = jnp.where(v0, w0, 0.0)
    w1m = jnp.where(v1, w1, 0.0)
    w0x_ref[...] = jnp.broadcast_to(w0m[:, None], (T, E))
    w1x_ref[...] = jnp.broadcast_to(w1m[:, None], (T, E))


def _router(x, rw):
    return pl.pallas_call(
        _router_body,
        out_shape=(
            jax.ShapeDtypeStruct((2, T), jnp.int32),
            jax.ShapeDtypeStruct((2, T), jnp.int32),
            jax.ShapeDtypeStruct((T, E), jnp.float32),
            jax.ShapeDtypeStruct((T, E), jnp.float32),
            jax.ShapeDtypeStruct((NBLK,), jnp.int32),
        ),
    )(x, rw)


_DSUB = 32
_DCH = T // NW // _DSUB


def _dispatch(x, dstw8):
    mesh = plsc.VectorSubcoreMesh(core_axis_name="c", subcore_axis_name="s")

    @functools.partial(
        pl.kernel,
        out_type=jax.ShapeDtypeStruct((ROWS + 8, D), jnp.float32),
        mesh=mesh,
        scratch_types=[
            pltpu.VMEM((_DSUB, D), jnp.float32),
            pltpu.VMEM((_DSUB, D), jnp.float32),
            pltpu.VMEM((_DCH, _DSUB), jnp.int32),
            pltpu.VMEM((_DCH, _DSUB), jnp.int32),
            pltpu.SemaphoreType.DMA,
            pltpu.SemaphoreType.DMA,
            pltpu.SemaphoreType.DMA,
            pltpu.SemaphoreType.DMA,
        ],
    )
    def k(x_hbm, dstw8_hbm, buf_hbm, rows0_v, rows1_v, idxa_v, idxb_v,
          sa0, sa1, sb0, sb1):
        rows = (rows0_v, rows1_v)
        sa = (sa0, sa1)
        sb = (sb0, sb1)
        cid = lax.axis_index("c")
        sid = lax.axis_index("s")
        wid = sid * NC + cid
        tb = wid * (T // NW)
        pltpu.sync_copy(dstw8_hbm.at[0, wid], idxa_v)
        pltpu.sync_copy(dstw8_hbm.at[1, wid], idxb_v)
        pltpu.sync_copy(x_hbm.at[pl.ds(tb, _DSUB)], rows[0])
        for c in range(_DCH):
            slot = c % 2
            cpa = pltpu.async_copy(rows[slot], buf_hbm.at[idxa_v.at[c]],
                                   sa[slot])
            cpb = pltpu.async_copy(rows[slot], buf_hbm.at[idxb_v.at[c]],
                                   sb[slot])
            if c + 1 < _DCH:
                pltpu.sync_copy(
                    x_hbm.at[pl.ds(tb + (c + 1) * _DSUB, _DSUB)],
                    rows[(c + 1) % 2])
            cpa.wait()
            cpb.wait()

    return k(x, dstw8)


def _mlp_body(eblk_ref, buf_ref, wg_ref, wu_ref, wd_ref, out_ref):
    xb = buf_ref[...]
    wg = wg_ref[0]
    wu = wu_ref[0]
    wd = wd_ref[0]
    g = jnp.dot(xb, wg, preferred_element_type=jnp.float32)
    u = jnp.dot(xb, wu, preferred_element_type=jnp.float32)
    h = (g * jax.nn.sigmoid(g)) * u
    out_ref[...] = jnp.dot(h, wd, preferred_element_type=jnp.float32)


def _mlp(eblk, buf, w_gate, w_up, w_down):
    grid_spec = pltpu.PrefetchScalarGridSpec(
        num_scalar_prefetch=1,
        grid=(NBLK,),
        in_specs=[
            pl.BlockSpec((BLK, D), lambda b, eb: (b, 0)),
            pl.BlockSpec((1, D, F), lambda b, eb: (eb[b], 0, 0)),
            pl.BlockSpec((1, D, F), lambda b, eb: (eb[b], 0, 0)),
            pl.BlockSpec((1, F, D), lambda b, eb: (eb[b], 0, 0)),
        ],
        out_specs=pl.BlockSpec((BLK, D), lambda b, eb: (b, 0)),
    )
    return pl.pallas_call(
        _mlp_body,
        grid_spec=grid_spec,
        out_shape=jax.ShapeDtypeStruct((ROWS, D), jnp.float32),
    )(eblk, buf, w_gate, w_up, w_down)


_CSUB = 16
_CSTEPS = T // NW // _CSUB
_TPW = T // NW


def _combine(out_buf, dstg, w0x, w1x):
    mesh = plsc.VectorSubcoreMesh(core_axis_name="c", subcore_axis_name="s")

    @functools.partial(
        pl.kernel,
        out_type=jax.ShapeDtypeStruct((T, D), jnp.float32),
        mesh=mesh,
        scratch_types=[
            pltpu.VMEM((_CSUB, D), jnp.float32),
            pltpu.VMEM((_CSUB, D), jnp.float32),
            pltpu.VMEM((_CSUB, D), jnp.float32),
            pltpu.VMEM((_CSUB, D), jnp.float32),
            pltpu.VMEM((_CSUB, D), jnp.float32),
            pltpu.VMEM((_TPW,), jnp.int32),
            pltpu.VMEM((_TPW,), jnp.int32),
            pltpu.VMEM((_TPW, E), jnp.float32),
            pltpu.VMEM((_TPW, E), jnp.float32),
            pltpu.SemaphoreType.DMA,
            pltpu.SemaphoreType.DMA,
            pltpu.SemaphoreType.DMA,
            pltpu.SemaphoreType.DMA,
            pltpu.SemaphoreType.DMA,
        ],
    )
    def k(out_hbm, dstg_hbm, w0x_hbm, w1x_hbm, y_hbm,
          a0_v, a1_v, b0_v, b1_v, y_v, i0_v, i1_v, w0_v, w1_v,
          sa0, sa1, sb0, sb1, sy):
        a_v = (a0_v, a1_v)
        b_v = (b0_v, b1_v)
        sa = (sa0, sa1)
        sb = (sb0, sb1)
        cid = lax.axis_index("c")
        sid = lax.axis_index("s")
        wid = sid * NC + cid
        tb = wid * _TPW
        pltpu.sync_copy(dstg_hbm.at[0, pl.ds(tb, _TPW)], i0_v)
        pltpu.sync_copy(dstg_hbm.at[1, pl.ds(tb, _TPW)], i1_v)
        pltpu.sync_copy(w0x_hbm.at[pl.ds(tb, _TPW)], w0_v)
        pltpu.sync_copy(w1x_hbm.at[pl.ds(tb, _TPW)], w1_v)

        def gather(c):
            slot = c % 2
            sl = pl.ds(c * _CSUB, _CSUB)
            ca = pltpu.async_copy(out_hbm.at[i0_v.at[sl]], a_v[slot], sa[slot])
            cb = pltpu.async_copy(out_hbm.at[i1_v.at[sl]], b_v[slot], sb[slot])
            return ca, cb

        pend = gather(0)
        ystore = None
        for c in range(_CSTEPS):
            slot = c % 2
            nxt = gather(c + 1) if c + 1 < _CSTEPS else None
            pend[0].wait()
            pend[1].wait()
            if ystore is not None:
                ystore.wait()

            def tok(j, _):
                wa = w0_v[c * _CSUB + j, :]
                wb = w1_v[c * _CSUB + j, :]
                for v in range(D // 16):
                    sl = pl.ds(v * 16, 16)
                    y_v[j, sl] = (a_v[slot][j, sl] * wa
                                  + b_v[slot][j, sl] * wb)
                return 0

            lax.fori_loop(0, _CSUB, tok, 0)
            ystore = pltpu.async_copy(
                y_v, y_hbm.at[pl.ds(tb + c * _CSUB, _CSUB)], sy)
            pend = nxt
        ystore.wait()

    return k(out_buf, dstg, w0x, w1x)


def kernel(hidden_states, router_w, w_gate, w_up, w_down):
    dstw, dstg, w0x, w1x, eblk = _router(hidden_states, router_w)
    dstw8 = dstw.reshape(2, NW, _DCH, _DSUB)
    buf = _dispatch(hidden_states, dstw8)
    out_buf = _mlp(eblk, buf, w_gate, w_up, w_down)
    return _combine(out_buf, dstg, w0x, w1x)
